# baseline (speedup 1.0000x reference)
import functools
import numpy as np
import jax
import jax.numpy as jnp
from jax import lax
from jax.experimental import pallas as pl
from jax.experimental.pallas import tpu as pltpu
from jax.experimental.pallas import tpu_sc as plsc

_L = 16
_TN = 10000
_D = 128
_B = 4
_N = 2500
_LAT = 128
_E = 320000

_S_BLK = 2000
_NBLK = _TN // _S_BLK

_NC = 2
_NS = 16
_NW = _NC * _NS
_EPW = _E // _NW
_CHUNK = 96
_ITERS = 104
_ETAIL = _EPW - _ITERS * _CHUNK
_RPT = 624
_RTAIL = _TN - _NS * _RPT


def _pe_const():
    pos = np.arange(_L, dtype=np.float64)[:, None]
    index = np.arange(_D, dtype=np.float64)[None, :]
    pe = pos / np.power(10000.0, (index - index % 2) / np.float32(_D))
    pe[:, 0::2] = np.sin(pe[:, 0::2])
    pe[:, 1::2] = np.cos(pe[:, 1::2])
    return jnp.asarray(pe.astype(np.float32))


def _leaky(x):
    return jnp.where(x >= 0, x, 0.2 * x)


def _r16(x):
    return x.astype(jnp.bfloat16).astype(jnp.float32)


def _wk_body(seq_ref, w1_ref, pe_ref, scal_ref, out_ref, acc_ref):
    i = pl.program_id(0)

    @pl.when(i == 0)
    def _():
        acc_ref[...] = jnp.zeros_like(acc_ref)

    sb = seq_ref[...]
    wb = w1_ref[...]
    pe = pe_ref[...]
    a16 = _r16(sb + pe[:, None, :])
    w16 = _r16(wb)
    acc_ref[...] += jnp.sum(a16 * w16[None, :, :], axis=1)

    @pl.when(i == _NBLK - 1)
    def _():
        c1b = scal_ref[0]
        c2w = scal_ref[1]
        c2b = scal_ref[2]
        c3w = scal_ref[3]
        c3b = scal_ref[4]
        g = scal_ref[5]
        be = scal_ref[6]
        mu = scal_ref[7]
        va = scal_ref[8]
        y1 = jnp.sum(acc_ref[...], axis=1, keepdims=True) + c1b
        y2 = c2w * y1 + c2b
        y3 = c3w * (y1 + y2) + c3b
        t = y1 + y2 + y3
        t = g * (t - mu) / jnp.sqrt(va + 1e-3) + be
        t = _leaky(t)
        m = jnp.max(t, axis=0, keepdims=True)
        ex = jnp.exp(t - m)
        w = ex / jnp.sum(ex, axis=0, keepdims=True)
        out_ref[...] = jnp.broadcast_to(w, (_L, _D))


def _weights_tc(seq_graph, conv1_w, pe, scal):
    return pl.pallas_call(
        _wk_body,
        grid=(_NBLK,),
        in_specs=[
            pl.BlockSpec((_L, _S_BLK, _D), lambda i: (0, i, 0)),
            pl.BlockSpec((_S_BLK, _D), lambda i: (i, 0)),
            pl.BlockSpec((_L, _D), lambda i: (0, 0)),
            pl.BlockSpec(memory_space=pltpu.SMEM),
        ],
        out_specs=pl.BlockSpec((_L, _D), lambda i: (0, 0)),
        out_shape=jax.ShapeDtypeStruct((_L, _D), jnp.float32),
        scratch_shapes=[
            pltpu.VMEM((_L, _D), jnp.float32),
        ],
    )(seq_graph, conv1_w, pe, scal)


def _si_body(seq_ref, w_ref, pe_ref, out_ref):
    sb = seq_ref[...]
    w = w_ref[...]
    pw = jnp.sum(w * pe_ref[...], axis=0, keepdims=True)
    out_ref[...] = jnp.sum(sb * w[:, None, :], axis=0) + pw


def _seq_int_tc(seq_graph, wbc, pe):
    return pl.pallas_call(
        _si_body,
        grid=(_NBLK,),
        in_specs=[
            pl.BlockSpec((_L, _S_BLK, _D), lambda i: (0, i, 0)),
            pl.BlockSpec((_L, _D), lambda i: (0, 0)),
            pl.BlockSpec((_L, _D), lambda i: (0, 0)),
        ],
        out_specs=pl.BlockSpec((_S_BLK, _D), lambda i: (i, 0)),
        out_shape=jax.ShapeDtypeStruct((_TN, _D), jnp.float32),
    )(seq_graph, wbc, pe)


def _segsum_body(tbl_h, src_h, dstm_h, dstt_h, zer_h, out_h,
                 src_v, dst_v, dstt_v, rows_v, rowst_v, acc_sh, sem, ssem):
    c = lax.axis_index("c")
    s = lax.axis_index("s")
    wid = s * _NC + c
    pltpu.sync_copy(src_h.at[wid], src_v)
    pltpu.sync_copy(dstm_h.at[wid], dst_v)
    pltpu.sync_copy(dstt_h.at[wid], dstt_v)
    pltpu.sync_copy(zer_h.at[pl.ds(s * _RPT, _RPT)],
                    acc_sh.at[pl.ds(s * _RPT, _RPT)])

    @pl.when(s == _NS - 1)
    def _():
        pltpu.sync_copy(zer_h.at[pl.ds(_NS * _RPT, _RTAIL)],
                        acc_sh.at[pl.ds(_NS * _RPT, _RTAIL)])

    plsc.subcore_barrier()

    pltpu.async_copy(
        tbl_h.at[src_v.at[pl.ds(_ITERS * _CHUNK, _ETAIL)]],
        rowst_v, sem).wait()
    pltpu.sync_copy(rowst_v, acc_sh.at[dstt_v], add=True)

    def _gather(j, par):
        return pltpu.make_async_copy(
            tbl_h.at[src_v.at[pl.ds(j * _CHUNK, _CHUNK)]],
            rows_v.at[par], sem)

    def _scatter_start(j, par):
        pltpu.async_copy(rows_v.at[par], acc_sh.at[dst_v.at[j]],
                         ssem.at[par], add=True)

    def _scatter_wait(j, par):
        pltpu.make_async_copy(rows_v.at[par], acc_sh.at[dst_v.at[j]],
                              ssem.at[par]).wait()

    _gather(0, 0).start()

    def body(j, par):
        _gather(j, par).wait()
        _scatter_start(j, par)

        @pl.when(j < _ITERS - 1)
        def _():
            @pl.when(j > 0)
            def _():
                _scatter_wait(j - 1, 1 - par)

            _gather(j + 1, 1 - par).start()

        return 1 - par

    lax.fori_loop(0, _ITERS, body, 0)
    _scatter_wait(_ITERS - 2, 0)
    _scatter_wait(_ITERS - 1, 1)
    plsc.subcore_barrier()
    pltpu.sync_copy(acc_sh.at[pl.ds(s * _RPT, _RPT)],
                    out_h.at[c, pl.ds(s * _RPT, _RPT)])

    @pl.when(s == _NS - 1)
    def _():
        pltpu.sync_copy(acc_sh.at[pl.ds(_NS * _RPT, _RTAIL)],
                        out_h.at[c, pl.ds(_NS * _RPT, _RTAIL)])


def _segsum_sc(seq_int, src, dst, zeros):
    mesh = plsc.VectorSubcoreMesh(core_axis_name="c", subcore_axis_name="s")
    f = functools.partial(
        pl.kernel,
        mesh=mesh,
        out_type=jax.ShapeDtypeStruct((_NC, _TN, _D), jnp.float32),
        scratch_types=[
            pltpu.VMEM((_EPW,), jnp.int32),
            pltpu.VMEM((_ITERS, _CHUNK), jnp.int32),
            pltpu.VMEM((_ETAIL,), jnp.int32),
            pltpu.VMEM((2, _CHUNK, _D), jnp.float32),
            pltpu.VMEM((_ETAIL, _D), jnp.float32),
            pltpu.VMEM_SHARED((_TN, _D), jnp.float32),
            pltpu.SemaphoreType.DMA,
            pltpu.SemaphoreType.DMA((2,)),
        ],
    )(_segsum_body)
    d2 = dst.reshape(_NW, _EPW)
    dst_main = d2[:, :_ITERS * _CHUNK].reshape(_NW, _ITERS, _CHUNK)
    dst_tail = d2[:, _ITERS * _CHUNK:]
    return f(seq_int, src.reshape(_NW, _EPW), dst_main, dst_tail, zeros)


def _fin_body(x_ref, ap_ref, g1w_ref, g1b_ref, wgg_ref, wgb_ref, wgm_ref,
              wgv_ref, g2w_ref, lev_ref, lw_ref, outw_ref, outb_ref,
              scal_ref, out_ref):
    eps1 = scal_ref[0]
    eps2 = scal_ref[1]
    g2b = scal_ref[2]
    igg = scal_ref[3]
    igb = scal_ref[4]
    igm = scal_ref[5]
    igv = scal_ref[6]
    lapb = scal_ref[7]

    agg = ap_ref[0] + ap_ref[1]
    x = x_ref[...]
    g1w = _r16(g1w_ref[...])
    g2w = _r16(g2w_ref[...])
    lap = (jnp.sum(_r16(lev_ref[...]) * _r16(lw_ref[...]), axis=1,
                   keepdims=True) + lapb)
    wg_div = jnp.sqrt(wgv_ref[...] + 1e-3)

    rows = []
    for b in range(_B):
        h1 = _r16((1.0 + eps1) * x[b] + agg[b])
        wgp = lax.dot_general(h1, g1w, (((1,), (0,)), ((), ())),
                              preferred_element_type=jnp.float32)
        wgp = wgp + g1b_ref[...]
        a = _leaky(wgg_ref[...] * (wgp - wgm_ref[...]) / wg_div
                   + wgb_ref[...])
        m = jnp.max(a, axis=0, keepdims=True)
        e = jnp.exp(a - m)
        h2 = _r16((1.0 + eps2) * x[b] + agg[b])
        sip = jnp.sum(h2 * g2w, axis=1, keepdims=True) + g2b
        si = _leaky(igg * (sip - igm) / jnp.sqrt(igv + 1e-3) + igb)
        v = si + lap
        num = jnp.sum(e * v, axis=0, keepdims=True)
        den = jnp.sum(e, axis=0, keepdims=True)
        rows.append(num / den)
    integral = jnp.concatenate(rows, axis=0)
    out = lax.dot_general(_r16(integral), _r16(outw_ref[...]),
                          (((1,), (0,)), ((), ())),
                          preferred_element_type=jnp.float32)
    out_ref[...] = out + outb_ref[...]


def _final_tc(x, agg_parts, g1w, g1b, wgg, wgb, wgm, wgv, g2w, lev, lw,
              outw, outb, scal):
    return pl.pallas_call(
        _fin_body,
        in_specs=[
            pl.BlockSpec(),
            pl.BlockSpec(),
            pl.BlockSpec(),
            pl.BlockSpec(),
            pl.BlockSpec(),
            pl.BlockSpec(),
            pl.BlockSpec(),
            pl.BlockSpec(),
            pl.BlockSpec(),
            pl.BlockSpec(),
            pl.BlockSpec(),
            pl.BlockSpec(),
            pl.BlockSpec(),
            pl.BlockSpec(memory_space=pltpu.SMEM),
        ],
        out_specs=pl.BlockSpec(),
        out_shape=jax.ShapeDtypeStruct((_B, _LAT), jnp.float32),
    )(x, agg_parts, g1w, g1b, wgg, wgb, wgm, wgv, g2w, lev, lw, outw,
      outb, scal)


def kernel(seq_graph, e_index, conv1_w, conv1_b, conv2_w, conv2_b, conv3_w,
           conv3_b, cn_gamma, cn_beta, cn_mean, cn_var, gin1_eps, gin1_w,
           gin1_b, wg_gamma, wg_beta, wg_mean, wg_var, gin2_eps, gin2_w,
           gin2_b, ig_gamma, ig_beta, ig_mean, ig_var, lap_eigvec, lap_w,
           lap_b, out_w, out_b):
    pe = _pe_const()

    scal_a = jnp.concatenate([
        conv1_b, conv2_w, conv2_b, conv3_w, conv3_b,
        cn_gamma, cn_beta, cn_mean, cn_var,
    ]).astype(jnp.float32)

    wbc = _weights_tc(seq_graph, conv1_w, pe, scal_a)
    seq_int = _seq_int_tc(seq_graph, wbc, pe)

    src = e_index[0]
    dst = e_index[1]
    zeros = jnp.zeros((_TN, _D), jnp.float32)
    agg_parts = _segsum_sc(seq_int, src, dst, zeros)

    scal_d = jnp.stack([
        gin1_eps, gin2_eps, gin2_b[0], ig_gamma[0], ig_beta[0],
        ig_mean[0], ig_var[0], lap_b[0],
    ]).astype(jnp.float32)

    x4 = seq_int.reshape(_B, _N, _D)
    ap4 = agg_parts.reshape(_NC, _B, _N, _D)

    out = _final_tc(
        x4, ap4, gin1_w,
        gin1_b.reshape(1, _LAT),
        wg_gamma.reshape(1, _LAT), wg_beta.reshape(1, _LAT),
        wg_mean.reshape(1, _LAT), wg_var.reshape(1, _LAT),
        gin2_w.reshape(1, _D),
        lap_eigvec, lap_w.reshape(1, -1),
        out_w, out_b.reshape(1, _LAT),
        scal_d,
    )
    return out


# speedup vs baseline: 1.0082x; 1.0082x over previous
import functools
import numpy as np
import jax
import jax.numpy as jnp
from jax import lax
from jax.experimental import pallas as pl
from jax.experimental.pallas import tpu as pltpu
from jax.experimental.pallas import tpu_sc as plsc

_L = 16
_TN = 10000
_D = 128
_B = 4
_N = 2500
_LAT = 128
_E = 320000

_S_BLK = 1000
_NBLK = _TN // _S_BLK

_NC = 2
_NS = 16
_NW = _NC * _NS
_EPW = _E // _NW
_CHUNK = 96
_ITERS = 104
_ETAIL = _EPW - _ITERS * _CHUNK
_RPT = 624
_RTAIL = _TN - _NS * _RPT


def _pe_const():
    pos = np.arange(_L, dtype=np.float64)[:, None]
    index = np.arange(_D, dtype=np.float64)[None, :]
    pe = pos / np.power(10000.0, (index - index % 2) / np.float32(_D))
    pe[:, 0::2] = np.sin(pe[:, 0::2])
    pe[:, 1::2] = np.cos(pe[:, 1::2])
    return jnp.asarray(pe.astype(np.float32))


def _leaky(x):
    return jnp.where(x >= 0, x, 0.2 * x)


def _r16(x):
    return x.astype(jnp.bfloat16).astype(jnp.float32)


def _wk_body(seq_ref, w1_ref, pe_ref, scal_ref, out_ref, acc_ref):
    i = pl.program_id(0)

    @pl.when(i == 0)
    def _():
        acc_ref[...] = jnp.zeros_like(acc_ref)

    sb = seq_ref[...]
    wb = w1_ref[...]
    pe = pe_ref[...]
    a16 = _r16(sb + pe[:, None, :])
    w16 = _r16(wb)
    acc_ref[...] += jnp.sum(a16 * w16[None, :, :], axis=1)

    @pl.when(i == _NBLK - 1)
    def _():
        c1b = scal_ref[0]
        c2w = scal_ref[1]
        c2b = scal_ref[2]
        c3w = scal_ref[3]
        c3b = scal_ref[4]
        g = scal_ref[5]
        be = scal_ref[6]
        mu = scal_ref[7]
        va = scal_ref[8]
        y1 = jnp.sum(acc_ref[...], axis=1, keepdims=True) + c1b
        y2 = c2w * y1 + c2b
        y3 = c3w * (y1 + y2) + c3b
        t = y1 + y2 + y3
        t = g * (t - mu) / jnp.sqrt(va + 1e-3) + be
        t = _leaky(t)
        m = jnp.max(t, axis=0, keepdims=True)
        ex = jnp.exp(t - m)
        w = ex / jnp.sum(ex, axis=0, keepdims=True)
        out_ref[...] = jnp.broadcast_to(w, (_L, _D))


def _weights_tc(seq_graph, conv1_w, pe, scal):
    return pl.pallas_call(
        _wk_body,
        grid=(_NBLK,),
        in_specs=[
            pl.BlockSpec((_L, _S_BLK, _D), lambda i: (0, i, 0)),
            pl.BlockSpec((_S_BLK, _D), lambda i: (i, 0)),
            pl.BlockSpec((_L, _D), lambda i: (0, 0)),
            pl.BlockSpec(memory_space=pltpu.SMEM),
        ],
        out_specs=pl.BlockSpec((_L, _D), lambda i: (0, 0)),
        out_shape=jax.ShapeDtypeStruct((_L, _D), jnp.float32),
        scratch_shapes=[
            pltpu.VMEM((_L, _D), jnp.float32),
        ],
    )(seq_graph, conv1_w, pe, scal)


def _si_body(seq_ref, w_ref, pe_ref, out_ref):
    sb = seq_ref[...]
    w = w_ref[...]
    pw = jnp.sum(w * pe_ref[...], axis=0, keepdims=True)
    out_ref[...] = jnp.sum(sb * w[:, None, :], axis=0) + pw


def _seq_int_tc(seq_graph, wbc, pe):
    return pl.pallas_call(
        _si_body,
        grid=(_NBLK,),
        in_specs=[
            pl.BlockSpec((_L, _S_BLK, _D), lambda i: (0, i, 0)),
            pl.BlockSpec((_L, _D), lambda i: (0, 0)),
            pl.BlockSpec((_L, _D), lambda i: (0, 0)),
        ],
        out_specs=pl.BlockSpec((_S_BLK, _D), lambda i: (i, 0)),
        out_shape=jax.ShapeDtypeStruct((_TN, _D), jnp.float32),
    )(seq_graph, wbc, pe)


def _segsum_body(tbl_h, src_h, dstm_h, dstt_h, zer_h, out_h,
                 src_v, dst_v, dstt_v, rows_v, rowst_v, acc_sh, sem, ssem):
    c = lax.axis_index("c")
    s = lax.axis_index("s")
    wid = s * _NC + c
    pltpu.sync_copy(src_h.at[wid], src_v)
    pltpu.sync_copy(dstm_h.at[wid], dst_v)
    pltpu.sync_copy(dstt_h.at[wid], dstt_v)
    pltpu.sync_copy(zer_h.at[pl.ds(s * _RPT, _RPT)],
                    acc_sh.at[pl.ds(s * _RPT, _RPT)])

    @pl.when(s == _NS - 1)
    def _():
        pltpu.sync_copy(zer_h.at[pl.ds(_NS * _RPT, _RTAIL)],
                        acc_sh.at[pl.ds(_NS * _RPT, _RTAIL)])

    plsc.subcore_barrier()

    pltpu.async_copy(
        tbl_h.at[src_v.at[pl.ds(_ITERS * _CHUNK, _ETAIL)]],
        rowst_v, sem).wait()
    pltpu.sync_copy(rowst_v, acc_sh.at[dstt_v], add=True)

    def _gather(j, par):
        return pltpu.make_async_copy(
            tbl_h.at[src_v.at[pl.ds(j * _CHUNK, _CHUNK)]],
            rows_v.at[par], sem)

    def _scatter_start(j, par):
        pltpu.async_copy(rows_v.at[par], acc_sh.at[dst_v.at[j]],
                         ssem.at[par], add=True)

    def _scatter_wait(j, par):
        pltpu.make_async_copy(rows_v.at[par], acc_sh.at[dst_v.at[j]],
                              ssem.at[par]).wait()

    _gather(0, 0).start()

    def body(j, par):
        _gather(j, par).wait()
        _scatter_start(j, par)

        @pl.when(j < _ITERS - 1)
        def _():
            @pl.when(j > 0)
            def _():
                _scatter_wait(j - 1, 1 - par)

            _gather(j + 1, 1 - par).start()

        return 1 - par

    lax.fori_loop(0, _ITERS, body, 0)
    _scatter_wait(_ITERS - 2, 0)
    _scatter_wait(_ITERS - 1, 1)
    plsc.subcore_barrier()
    pltpu.sync_copy(acc_sh.at[pl.ds(s * _RPT, _RPT)],
                    out_h.at[c, pl.ds(s * _RPT, _RPT)])

    @pl.when(s == _NS - 1)
    def _():
        pltpu.sync_copy(acc_sh.at[pl.ds(_NS * _RPT, _RTAIL)],
                        out_h.at[c, pl.ds(_NS * _RPT, _RTAIL)])


def _segsum_sc(seq_int, src, dst, zeros):
    mesh = plsc.VectorSubcoreMesh(core_axis_name="c", subcore_axis_name="s")
    f = functools.partial(
        pl.kernel,
        mesh=mesh,
        out_type=jax.ShapeDtypeStruct((_NC, _TN, _D), jnp.float32),
        scratch_types=[
            pltpu.VMEM((_EPW,), jnp.int32),
            pltpu.VMEM((_ITERS, _CHUNK), jnp.int32),
            pltpu.VMEM((_ETAIL,), jnp.int32),
            pltpu.VMEM((2, _CHUNK, _D), jnp.float32),
            pltpu.VMEM((_ETAIL, _D), jnp.float32),
            pltpu.VMEM_SHARED((_TN, _D), jnp.float32),
            pltpu.SemaphoreType.DMA,
            pltpu.SemaphoreType.DMA((2,)),
        ],
    )(_segsum_body)
    d2 = dst.reshape(_NW, _EPW)
    dst_main = d2[:, :_ITERS * _CHUNK].reshape(_NW, _ITERS, _CHUNK)
    dst_tail = d2[:, _ITERS * _CHUNK:]
    return f(seq_int, src.reshape(_NW, _EPW), dst_main, dst_tail, zeros)


def _fin_body(x_ref, ap_ref, g1w_ref, g1b_ref, wgg_ref, wgb_ref, wgm_ref,
              wgv_ref, g2w_ref, lev_ref, lw_ref, outw_ref, outb_ref,
              scal_ref, out_ref):
    eps1 = scal_ref[0]
    eps2 = scal_ref[1]
    g2b = scal_ref[2]
    igg = scal_ref[3]
    igb = scal_ref[4]
    igm = scal_ref[5]
    igv = scal_ref[6]
    lapb = scal_ref[7]

    agg = ap_ref[0] + ap_ref[1]
    x = x_ref[...]
    g1w = _r16(g1w_ref[...])
    g2w = _r16(g2w_ref[...])
    lap = (jnp.sum(_r16(lev_ref[...]) * _r16(lw_ref[...]), axis=1,
                   keepdims=True) + lapb)
    wg_div = jnp.sqrt(wgv_ref[...] + 1e-3)

    rows = []
    for b in range(_B):
        h1 = _r16((1.0 + eps1) * x[b] + agg[b])
        wgp = lax.dot_general(h1, g1w, (((1,), (0,)), ((), ())),
                              preferred_element_type=jnp.float32)
        wgp = wgp + g1b_ref[...]
        a = _leaky(wgg_ref[...] * (wgp - wgm_ref[...]) / wg_div
                   + wgb_ref[...])
        m = jnp.max(a, axis=0, keepdims=True)
        e = jnp.exp(a - m)
        h2 = _r16((1.0 + eps2) * x[b] + agg[b])
        sip = jnp.sum(h2 * g2w, axis=1, keepdims=True) + g2b
        si = _leaky(igg * (sip - igm) / jnp.sqrt(igv + 1e-3) + igb)
        v = si + lap
        num = jnp.sum(e * v, axis=0, keepdims=True)
        den = jnp.sum(e, axis=0, keepdims=True)
        rows.append(num / den)
    integral = jnp.concatenate(rows, axis=0)
    out = lax.dot_general(_r16(integral), _r16(outw_ref[...]),
                          (((1,), (0,)), ((), ())),
                          preferred_element_type=jnp.float32)
    out_ref[...] = out + outb_ref[...]


def _final_tc(x, agg_parts, g1w, g1b, wgg, wgb, wgm, wgv, g2w, lev, lw,
              outw, outb, scal):
    return pl.pallas_call(
        _fin_body,
        in_specs=[
            pl.BlockSpec(),
            pl.BlockSpec(),
            pl.BlockSpec(),
            pl.BlockSpec(),
            pl.BlockSpec(),
            pl.BlockSpec(),
            pl.BlockSpec(),
            pl.BlockSpec(),
            pl.BlockSpec(),
            pl.BlockSpec(),
            pl.BlockSpec(),
            pl.BlockSpec(),
            pl.BlockSpec(),
            pl.BlockSpec(memory_space=pltpu.SMEM),
        ],
        out_specs=pl.BlockSpec(),
        out_shape=jax.ShapeDtypeStruct((_B, _LAT), jnp.float32),
    )(x, agg_parts, g1w, g1b, wgg, wgb, wgm, wgv, g2w, lev, lw, outw,
      outb, scal)


def kernel(seq_graph, e_index, conv1_w, conv1_b, conv2_w, conv2_b, conv3_w,
           conv3_b, cn_gamma, cn_beta, cn_mean, cn_var, gin1_eps, gin1_w,
           gin1_b, wg_gamma, wg_beta, wg_mean, wg_var, gin2_eps, gin2_w,
           gin2_b, ig_gamma, ig_beta, ig_mean, ig_var, lap_eigvec, lap_w,
           lap_b, out_w, out_b):
    pe = _pe_const()

    scal_a = jnp.concatenate([
        conv1_b, conv2_w, conv2_b, conv3_w, conv3_b,
        cn_gamma, cn_beta, cn_mean, cn_var,
    ]).astype(jnp.float32)

    wbc = _weights_tc(seq_graph, conv1_w, pe, scal_a)
    seq_int = _seq_int_tc(seq_graph, wbc, pe)

    src = e_index[0]
    dst = e_index[1]
    zeros = jnp.zeros((_TN, _D), jnp.float32)
    agg_parts = _segsum_sc(seq_int, src, dst, zeros)

    scal_d = jnp.stack([
        gin1_eps, gin2_eps, gin2_b[0], ig_gamma[0], ig_beta[0],
        ig_mean[0], ig_var[0], lap_b[0],
    ]).astype(jnp.float32)

    x4 = seq_int.reshape(_B, _N, _D)
    ap4 = agg_parts.reshape(_NC, _B, _N, _D)

    out = _final_tc(
        x4, ap4, gin1_w,
        gin1_b.reshape(1, _LAT),
        wg_gamma.reshape(1, _LAT), wg_beta.reshape(1, _LAT),
        wg_mean.reshape(1, _LAT), wg_var.reshape(1, _LAT),
        gin2_w.reshape(1, _D),
        lap_eigvec, lap_w.reshape(1, -1),
        out_w, out_b.reshape(1, _LAT),
        scal_d,
    )
    return out


# speedup vs baseline: 1.0125x; 1.0043x over previous
import functools
import numpy as np
import jax
import jax.numpy as jnp
from jax import lax
from jax.experimental import pallas as pl
from jax.experimental.pallas import tpu as pltpu
from jax.experimental.pallas import tpu_sc as plsc

_L = 16
_TN = 10000
_D = 128
_B = 4
_N = 2500
_LAT = 128
_E = 320000

_S_BLK = 1000
_NBLK = _TN // _S_BLK

_NC = 2
_NS = 16
_NW = _NC * _NS
_EPW = _E // _NW
_CHUNK = 96
_ITERS = 104
_ETAIL = _EPW - _ITERS * _CHUNK
_RPT = 624
_RTAIL = _TN - _NS * _RPT


def _pe_const():
    pos = np.arange(_L, dtype=np.float64)[:, None]
    index = np.arange(_D, dtype=np.float64)[None, :]
    pe = pos / np.power(10000.0, (index - index % 2) / np.float32(_D))
    pe[:, 0::2] = np.sin(pe[:, 0::2])
    pe[:, 1::2] = np.cos(pe[:, 1::2])
    return jnp.asarray(pe.astype(np.float32))


def _leaky(x):
    return jnp.where(x >= 0, x, 0.2 * x)


def _r16(x):
    return x.astype(jnp.bfloat16).astype(jnp.float32)


def _wk_body(seq_ref, w1_ref, pe_ref, scal_ref, out_ref, acc_ref):
    i = pl.program_id(0)

    @pl.when(i == 0)
    def _():
        acc_ref[...] = jnp.zeros_like(acc_ref)

    sb = seq_ref[...]
    wb = w1_ref[...]
    pe = pe_ref[...]
    a16 = _r16(sb + pe[:, None, :])
    w16 = _r16(wb)
    acc_ref[...] += jnp.sum(a16 * w16[None, :, :], axis=1)

    @pl.when(i == _NBLK - 1)
    def _():
        c1b = scal_ref[0]
        c2w = scal_ref[1]
        c2b = scal_ref[2]
        c3w = scal_ref[3]
        c3b = scal_ref[4]
        g = scal_ref[5]
        be = scal_ref[6]
        mu = scal_ref[7]
        va = scal_ref[8]
        y1 = jnp.sum(acc_ref[...], axis=1, keepdims=True) + c1b
        y2 = c2w * y1 + c2b
        y3 = c3w * (y1 + y2) + c3b
        t = y1 + y2 + y3
        t = g * (t - mu) / jnp.sqrt(va + 1e-3) + be
        t = _leaky(t)
        m = jnp.max(t, axis=0, keepdims=True)
        ex = jnp.exp(t - m)
        w = ex / jnp.sum(ex, axis=0, keepdims=True)
        out_ref[...] = jnp.broadcast_to(w, (_L, _D))


def _weights_tc(seq_graph, conv1_w, pe, scal):
    return pl.pallas_call(
        _wk_body,
        grid=(_NBLK,),
        in_specs=[
            pl.BlockSpec((_L, _S_BLK, _D), lambda i: (0, i, 0)),
            pl.BlockSpec((_S_BLK, _D), lambda i: (i, 0)),
            pl.BlockSpec((_L, _D), lambda i: (0, 0)),
            pl.BlockSpec(memory_space=pltpu.SMEM),
        ],
        out_specs=pl.BlockSpec((_L, _D), lambda i: (0, 0)),
        out_shape=jax.ShapeDtypeStruct((_L, _D), jnp.float32),
        scratch_shapes=[
            pltpu.VMEM((_L, _D), jnp.float32),
        ],
    )(seq_graph, conv1_w, pe, scal)


def _si_body(seq_ref, w_ref, pe_ref, out_ref):
    sb = seq_ref[...]
    w = w_ref[...]
    pw = jnp.sum(w * pe_ref[...], axis=0, keepdims=True)
    out_ref[...] = jnp.sum(sb * w[:, None, :], axis=0) + pw


def _seq_int_tc(seq_graph, wbc, pe):
    return pl.pallas_call(
        _si_body,
        grid=(_NBLK,),
        in_specs=[
            pl.BlockSpec((_L, _S_BLK, _D), lambda i: (0, i, 0)),
            pl.BlockSpec((_L, _D), lambda i: (0, 0)),
            pl.BlockSpec((_L, _D), lambda i: (0, 0)),
        ],
        out_specs=pl.BlockSpec((_S_BLK, _D), lambda i: (i, 0)),
        out_shape=jax.ShapeDtypeStruct((_TN, _D), jnp.float32),
    )(seq_graph, wbc, pe)


def _segsum_body(tbl_h, src_h, dstm_h, dstt_h, zer_h, out_h,
                 src_v, dst_v, dstt_v, rows_v, rowst_v, acc_sh, sem, ssem):
    c = lax.axis_index("c")
    s = lax.axis_index("s")
    wid = s * _NC + c
    pltpu.sync_copy(src_h.at[wid], src_v)
    pltpu.sync_copy(dstm_h.at[wid], dst_v)
    pltpu.sync_copy(dstt_h.at[wid], dstt_v)
    pltpu.sync_copy(zer_h.at[pl.ds(s * _RPT, _RPT)],
                    acc_sh.at[pl.ds(s * _RPT, _RPT)])

    @pl.when(s == _NS - 1)
    def _():
        pltpu.sync_copy(zer_h.at[pl.ds(_NS * _RPT, _RTAIL)],
                        acc_sh.at[pl.ds(_NS * _RPT, _RTAIL)])

    plsc.subcore_barrier()

    pltpu.async_copy(
        tbl_h.at[src_v.at[pl.ds(_ITERS * _CHUNK, _ETAIL)]],
        rowst_v, sem).wait()
    pltpu.sync_copy(rowst_v, acc_sh.at[dstt_v], add=True)

    def _gather(j, par):
        return pltpu.make_async_copy(
            tbl_h.at[src_v.at[pl.ds(j * _CHUNK, _CHUNK)]],
            rows_v.at[par], sem)

    def _scatter_start(j, par):
        pltpu.async_copy(rows_v.at[par], acc_sh.at[dst_v.at[j]],
                         ssem.at[par], add=True)

    def _scatter_wait(j, par):
        pltpu.make_async_copy(rows_v.at[par], acc_sh.at[dst_v.at[j]],
                              ssem.at[par]).wait()

    _gather(0, 0).start()

    def body(i, carry):
        j0 = 2 * i
        j1 = j0 + 1
        _gather(j0, 0).wait()
        _scatter_start(j0, 0)

        @pl.when(i > 0)
        def _():
            _scatter_wait(j0 - 1, 1)

        _gather(j1, 1).start()
        _gather(j1, 1).wait()
        _scatter_start(j1, 1)
        _scatter_wait(j0, 0)

        @pl.when(j1 < _ITERS - 1)
        def _():
            _gather(j1 + 1, 0).start()

        return carry

    lax.fori_loop(0, _ITERS // 2, body, 0)
    _scatter_wait(_ITERS - 1, 1)
    plsc.subcore_barrier()
    pltpu.sync_copy(acc_sh.at[pl.ds(s * _RPT, _RPT)],
                    out_h.at[c, pl.ds(s * _RPT, _RPT)])

    @pl.when(s == _NS - 1)
    def _():
        pltpu.sync_copy(acc_sh.at[pl.ds(_NS * _RPT, _RTAIL)],
                        out_h.at[c, pl.ds(_NS * _RPT, _RTAIL)])


def _segsum_sc(seq_int, src, dst, zeros):
    mesh = plsc.VectorSubcoreMesh(core_axis_name="c", subcore_axis_name="s")
    f = functools.partial(
        pl.kernel,
        mesh=mesh,
        out_type=jax.ShapeDtypeStruct((_NC, _TN, _D), jnp.float32),
        scratch_types=[
            pltpu.VMEM((_EPW,), jnp.int32),
            pltpu.VMEM((_ITERS, _CHUNK), jnp.int32),
            pltpu.VMEM((_ETAIL,), jnp.int32),
            pltpu.VMEM((2, _CHUNK, _D), jnp.float32),
            pltpu.VMEM((_ETAIL, _D), jnp.float32),
            pltpu.VMEM_SHARED((_TN, _D), jnp.float32),
            pltpu.SemaphoreType.DMA,
            pltpu.SemaphoreType.DMA((2,)),
        ],
    )(_segsum_body)
    d2 = dst.reshape(_NW, _EPW)
    dst_main = d2[:, :_ITERS * _CHUNK].reshape(_NW, _ITERS, _CHUNK)
    dst_tail = d2[:, _ITERS * _CHUNK:]
    return f(seq_int, src.reshape(_NW, _EPW), dst_main, dst_tail, zeros)


def _fin_body(x_ref, ap_ref, g1w_ref, g1b_ref, wgg_ref, wgb_ref, wgm_ref,
              wgv_ref, g2w_ref, lev_ref, lw_ref, outw_ref, outb_ref,
              scal_ref, out_ref):
    eps1 = scal_ref[0]
    eps2 = scal_ref[1]
    g2b = scal_ref[2]
    igg = scal_ref[3]
    igb = scal_ref[4]
    igm = scal_ref[5]
    igv = scal_ref[6]
    lapb = scal_ref[7]

    agg = ap_ref[0] + ap_ref[1]
    x = x_ref[...]
    g1w = _r16(g1w_ref[...])
    g2w = _r16(g2w_ref[...])
    lap = (jnp.sum(_r16(lev_ref[...]) * _r16(lw_ref[...]), axis=1,
                   keepdims=True) + lapb)
    wg_div = jnp.sqrt(wgv_ref[...] + 1e-3)

    rows = []
    for b in range(_B):
        h1 = _r16((1.0 + eps1) * x[b] + agg[b])
        wgp = lax.dot_general(h1, g1w, (((1,), (0,)), ((), ())),
                              preferred_element_type=jnp.float32)
        wgp = wgp + g1b_ref[...]
        a = _leaky(wgg_ref[...] * (wgp - wgm_ref[...]) / wg_div
                   + wgb_ref[...])
        m = jnp.max(a, axis=0, keepdims=True)
        e = jnp.exp(a - m)
        h2 = _r16((1.0 + eps2) * x[b] + agg[b])
        sip = jnp.sum(h2 * g2w, axis=1, keepdims=True) + g2b
        si = _leaky(igg * (sip - igm) / jnp.sqrt(igv + 1e-3) + igb)
        v = si + lap
        num = jnp.sum(e * v, axis=0, keepdims=True)
        den = jnp.sum(e, axis=0, keepdims=True)
        rows.append(num / den)
    integral = jnp.concatenate(rows, axis=0)
    out = lax.dot_general(_r16(integral), _r16(outw_ref[...]),
                          (((1,), (0,)), ((), ())),
                          preferred_element_type=jnp.float32)
    out_ref[...] = out + outb_ref[...]


def _final_tc(x, agg_parts, g1w, g1b, wgg, wgb, wgm, wgv, g2w, lev, lw,
              outw, outb, scal):
    return pl.pallas_call(
        _fin_body,
        in_specs=[
            pl.BlockSpec(),
            pl.BlockSpec(),
            pl.BlockSpec(),
            pl.BlockSpec(),
            pl.BlockSpec(),
            pl.BlockSpec(),
            pl.BlockSpec(),
            pl.BlockSpec(),
            pl.BlockSpec(),
            pl.BlockSpec(),
            pl.BlockSpec(),
            pl.BlockSpec(),
            pl.BlockSpec(),
            pl.BlockSpec(memory_space=pltpu.SMEM),
        ],
        out_specs=pl.BlockSpec(),
        out_shape=jax.ShapeDtypeStruct((_B, _LAT), jnp.float32),
    )(x, agg_parts, g1w, g1b, wgg, wgb, wgm, wgv, g2w, lev, lw, outw,
      outb, scal)


def kernel(seq_graph, e_index, conv1_w, conv1_b, conv2_w, conv2_b, conv3_w,
           conv3_b, cn_gamma, cn_beta, cn_mean, cn_var, gin1_eps, gin1_w,
           gin1_b, wg_gamma, wg_beta, wg_mean, wg_var, gin2_eps, gin2_w,
           gin2_b, ig_gamma, ig_beta, ig_mean, ig_var, lap_eigvec, lap_w,
           lap_b, out_w, out_b):
    pe = _pe_const()

    scal_a = jnp.concatenate([
        conv1_b, conv2_w, conv2_b, conv3_w, conv3_b,
        cn_gamma, cn_beta, cn_mean, cn_var,
    ]).astype(jnp.float32)

    wbc = _weights_tc(seq_graph, conv1_w, pe, scal_a)
    seq_int = _seq_int_tc(seq_graph, wbc, pe)

    src = e_index[0]
    dst = e_index[1]
    zeros = jnp.zeros((_TN, _D), jnp.float32)
    agg_parts = _segsum_sc(seq_int, src, dst, zeros)

    scal_d = jnp.stack([
        gin1_eps, gin2_eps, gin2_b[0], ig_gamma[0], ig_beta[0],
        ig_mean[0], ig_var[0], lap_b[0],
    ]).astype(jnp.float32)

    x4 = seq_int.reshape(_B, _N, _D)
    ap4 = agg_parts.reshape(_NC, _B, _N, _D)

    out = _final_tc(
        x4, ap4, gin1_w,
        gin1_b.reshape(1, _LAT),
        wg_gamma.reshape(1, _LAT), wg_beta.reshape(1, _LAT),
        wg_mean.reshape(1, _LAT), wg_var.reshape(1, _LAT),
        gin2_w.reshape(1, _D),
        lap_eigvec, lap_w.reshape(1, -1),
        out_w, out_b.reshape(1, _LAT),
        scal_d,
    )
    return out
